# baseline (speedup 1.0000x reference)
import functools

import jax
import jax.numpy as jnp
from jax import lax
from jax.experimental import pallas as pl
from jax.experimental.pallas import tpu as pltpu
from jax.experimental.pallas import tpu_sc as plsc

FREEZE_RATIO = 0.9

R = 16384
D = 1024
T = int(R * FREEZE_RATIO)
NC = 2
NS = 16
NW = NC * NS
LANES = 16
GRP = 8

GRP_LO = (T // GRP) * GRP
NZG = T - GRP_LO

NGROUPS = GRP_LO // GRP
GPW = NGROUPS // NW
ZPW = GPW * GRP
NEXTRA = NGROUPS - GPW * NW
EXTRA_LO = ZPW * NW

CH = 32
NFULL = ZPW // CH
ZTAIL = ZPW - NFULL * CH

COPY_LO = GRP_LO + GRP
NGROUPS_C = (R - COPY_LO) // GRP
GPW_C = NGROUPS_C // NW
NEXTRA_C = NGROUPS_C - GPW_C * NW
BASE_ROWS = GPW_C * GRP


_mesh = plsc.VectorSubcoreMesh(core_axis_name="c", subcore_axis_name="s")


@functools.partial(
    pl.kernel,
    mesh=_mesh,
    out_type=[
        jax.ShapeDtypeStruct((R, D), jnp.float32),
        jax.ShapeDtypeStruct((LANES,), jnp.int32),
    ],
    scratch_types=[
        pltpu.VMEM((CH, D), jnp.float32),
        pltpu.VMEM((BASE_ROWS + GRP, D), jnp.float32),
        pltpu.VMEM((GRP, D), jnp.float32),
        pltpu.VMEM((LANES,), jnp.int32),
        pltpu.SemaphoreType.DMA,
        pltpu.SemaphoreType.DMA,
        pltpu.SemaphoreType.DMA,
        pltpu.SemaphoreType.DMA,
    ],
)
def _freeze_sc(tokens_hbm, zsrc_hbm, out_hbm, cnt_hbm,
               zeros_v, buf_c, buf_m, cnt_v, sem_z, sem_i, sem_o, sem_zi):
    wid = lax.axis_index("s") * NC + lax.axis_index("c")

    copy_a = COPY_LO + (wid * GPW_C + jnp.minimum(wid, NEXTRA_C)) * GRP
    in_a = pltpu.async_copy(tokens_hbm.at[pl.ds(copy_a, BASE_ROWS)],
                            buf_c.at[pl.ds(0, BASE_ROWS)], sem_i)

    copy_b = copy_a + BASE_ROWS

    @pl.when(wid < NEXTRA_C)
    def _fire_in_b():
        pltpu.async_copy(tokens_hbm.at[pl.ds(copy_b, GRP)],
                         buf_c.at[pl.ds(BASE_ROWS, GRP)], sem_i)

    @pl.when(wid == NW - 1)
    def _fire_in_m():
        pltpu.async_copy(tokens_hbm.at[pl.ds(GRP_LO, GRP)], buf_m, sem_i)

    pltpu.async_copy(zsrc_hbm, zeros_v, sem_zi).wait()

    zbase = wid * ZPW
    z_handles = []
    for k in range(NFULL):
        z_handles.append(pltpu.async_copy(
            zeros_v, out_hbm.at[pl.ds(zbase + k * CH, CH)], sem_z))
    z_handles.append(pltpu.async_copy(
        zeros_v.at[pl.ds(0, ZTAIL)],
        out_hbm.at[pl.ds(zbase + NFULL * CH, ZTAIL)], sem_z))

    extra_lo = EXTRA_LO + wid * GRP

    @pl.when(wid < NEXTRA)
    def _fire_extra_zero():
        pltpu.async_copy(zeros_v.at[pl.ds(0, GRP)],
                         out_hbm.at[pl.ds(extra_lo, GRP)], sem_z).wait()

    in_a.wait()
    out_a = pltpu.async_copy(buf_c.at[pl.ds(0, BASE_ROWS)],
                             out_hbm.at[pl.ds(copy_a, BASE_ROWS)], sem_o)

    @pl.when(wid < NEXTRA_C)
    def _flush_b():
        pltpu.make_async_copy(tokens_hbm.at[pl.ds(copy_b, GRP)],
                              buf_c.at[pl.ds(BASE_ROWS, GRP)], sem_i).wait()
        pltpu.async_copy(buf_c.at[pl.ds(BASE_ROWS, GRP)],
                         out_hbm.at[pl.ds(copy_b, GRP)], sem_o).wait()

    @pl.when(wid == NW - 1)
    def _flush_m():
        pltpu.make_async_copy(tokens_hbm.at[pl.ds(GRP_LO, GRP)],
                              buf_m, sem_i).wait()

        def zero_col(c, carry):
            for r in range(NZG):
                buf_m[r, pl.ds(c * LANES, LANES)] = jnp.zeros(
                    (LANES,), jnp.float32)
            return carry

        lax.fori_loop(0, D // LANES, zero_col, 0)
        pltpu.async_copy(buf_m, out_hbm.at[pl.ds(GRP_LO, GRP)],
                         sem_o).wait()

    @pl.when(wid == 0)
    def _write_count():
        cnt_v[...] = jnp.full((LANES,), T, jnp.int32)
        pltpu.sync_copy(cnt_v, cnt_hbm)

    for h in z_handles:
        h.wait()
    out_a.wait()


@jax.jit
def kernel(tokens):
    b, s, d = tokens.shape
    flat = tokens.reshape(b * s, d)
    zsrc = jnp.zeros((CH, D), jnp.float32)
    out_flat, cnt = _freeze_sc(flat, zsrc)
    return out_flat.reshape(b, s, d), cnt[0]


# speedup vs baseline: 1.1416x; 1.1416x over previous
import functools

import jax
import jax.numpy as jnp
from jax import lax
from jax.experimental import pallas as pl
from jax.experimental.pallas import tpu as pltpu
from jax.experimental.pallas import tpu_sc as plsc

FREEZE_RATIO = 0.9

R = 16384
D = 1024
T = int(R * FREEZE_RATIO)
NC = 2
NS = 16
NW = NC * NS
LANES = 16
GRP = 8

GRP_LO = (T // GRP) * GRP
NZG = T - GRP_LO

NGROUPS = GRP_LO // GRP
GPW = NGROUPS // NW
ZPW = GPW * GRP
NEXTRA = NGROUPS - GPW * NW
EXTRA_LO = ZPW * NW

CH = 32
NFULL = ZPW // CH
ZTAIL = ZPW - NFULL * CH

COPY_LO = GRP_LO + GRP
NGROUPS_C = (R - COPY_LO) // GRP
GPW_C = NGROUPS_C // NW
NEXTRA_C = NGROUPS_C - GPW_C * NW
BASE_ROWS = GPW_C * GRP


_mesh = plsc.VectorSubcoreMesh(core_axis_name="c", subcore_axis_name="s")


@functools.partial(
    pl.kernel,
    mesh=_mesh,
    out_type=[
        jax.ShapeDtypeStruct((R, D), jnp.float32),
        jax.ShapeDtypeStruct((LANES,), jnp.int32),
    ],
    scratch_types=[
        pltpu.VMEM((CH, D), jnp.float32),
        pltpu.VMEM((BASE_ROWS + GRP, D), jnp.float32),
        pltpu.VMEM((GRP, D), jnp.float32),
        pltpu.VMEM((LANES,), jnp.int32),
        pltpu.SemaphoreType.DMA,
        pltpu.SemaphoreType.DMA,
        pltpu.SemaphoreType.DMA,
    ],
)
def _freeze_sc(tokens_hbm, out_hbm, cnt_hbm,
               zeros_v, buf_c, buf_m, cnt_v, sem_z, sem_i, sem_o):
    wid = lax.axis_index("s") * NC + lax.axis_index("c")

    copy_a = COPY_LO + (wid * GPW_C + jnp.minimum(wid, NEXTRA_C)) * GRP
    in_a = pltpu.async_copy(tokens_hbm.at[pl.ds(copy_a, BASE_ROWS)],
                            buf_c.at[pl.ds(0, BASE_ROWS)], sem_i)

    copy_b = copy_a + BASE_ROWS

    @pl.when(wid < NEXTRA_C)
    def _fire_in_b():
        pltpu.async_copy(tokens_hbm.at[pl.ds(copy_b, GRP)],
                         buf_c.at[pl.ds(BASE_ROWS, GRP)], sem_i)

    @pl.when(wid == NW - 1)
    def _fire_in_m():
        pltpu.async_copy(tokens_hbm.at[pl.ds(GRP_LO, GRP)], buf_m, sem_i)

    def fill_row(r, carry):
        for c in range(D // LANES):
            zeros_v[r, pl.ds(c * LANES, LANES)] = jnp.zeros(
                (LANES,), jnp.float32)
        return carry

    lax.fori_loop(0, CH, fill_row, 0)

    zbase = wid * ZPW
    z_handles = []
    for k in range(NFULL):
        z_handles.append(pltpu.async_copy(
            zeros_v, out_hbm.at[pl.ds(zbase + k * CH, CH)], sem_z))
    z_handles.append(pltpu.async_copy(
        zeros_v.at[pl.ds(0, ZTAIL)],
        out_hbm.at[pl.ds(zbase + NFULL * CH, ZTAIL)], sem_z))

    extra_lo = EXTRA_LO + wid * GRP

    @pl.when(wid < NEXTRA)
    def _fire_extra_zero():
        pltpu.async_copy(zeros_v.at[pl.ds(0, GRP)],
                         out_hbm.at[pl.ds(extra_lo, GRP)], sem_z).wait()

    in_a.wait()
    out_a = pltpu.async_copy(buf_c.at[pl.ds(0, BASE_ROWS)],
                             out_hbm.at[pl.ds(copy_a, BASE_ROWS)], sem_o)

    @pl.when(wid < NEXTRA_C)
    def _flush_b():
        pltpu.make_async_copy(tokens_hbm.at[pl.ds(copy_b, GRP)],
                              buf_c.at[pl.ds(BASE_ROWS, GRP)], sem_i).wait()
        pltpu.async_copy(buf_c.at[pl.ds(BASE_ROWS, GRP)],
                         out_hbm.at[pl.ds(copy_b, GRP)], sem_o).wait()

    @pl.when(wid == NW - 1)
    def _flush_m():
        pltpu.make_async_copy(tokens_hbm.at[pl.ds(GRP_LO, GRP)],
                              buf_m, sem_i).wait()

        def zero_col(c, carry):
            for r in range(NZG):
                buf_m[r, pl.ds(c * LANES, LANES)] = jnp.zeros(
                    (LANES,), jnp.float32)
            return carry

        lax.fori_loop(0, D // LANES, zero_col, 0)
        pltpu.async_copy(buf_m, out_hbm.at[pl.ds(GRP_LO, GRP)],
                         sem_o).wait()

    @pl.when(wid == 0)
    def _write_count():
        cnt_v[...] = jnp.full((LANES,), T, jnp.int32)
        pltpu.sync_copy(cnt_v, cnt_hbm)

    for h in z_handles:
        h.wait()
    out_a.wait()


@jax.jit
def kernel(tokens):
    b, s, d = tokens.shape
    flat = tokens.reshape(b * s, d)
    out_flat, cnt = _freeze_sc(flat)
    return out_flat.reshape(b, s, d), cnt[0]
